# baseline (speedup 1.0000x reference)
import functools

import jax
import jax.numpy as jnp
from jax import lax
from jax.experimental import pallas as pl
from jax.experimental.pallas import tpu as pltpu
from jax.experimental.pallas import tpu_sc as plsc

BATCH = 16384
EMBED = 16
_NC = 2
_NS = 16
_NW = _NC * _NS
_CHUNK = BATCH // _NW
_BLOCKS = _CHUNK // 16


def _body(uid_hbm, iid_hbm, uf_hbm, if_hbm, ub_hbm, ib_hbm, gb_hbm,
          pred_hbm, cvr_hbm,
          uid_v, iid_v, u_rows, i_rows, ub_v, ib_v, pred_v, cvr_v, gb_v,
          sem):
    wid = lax.axis_index("s") * _NC + lax.axis_index("c")
    base = wid * _CHUNK

    pltpu.sync_copy(uid_hbm.at[pl.ds(base, _CHUNK)], uid_v)
    pltpu.sync_copy(iid_hbm.at[pl.ds(base, _CHUNK)], iid_v)
    pltpu.sync_copy(gb_hbm, gb_v)

    cp_u = pltpu.async_copy(uf_hbm.at[uid_v], u_rows, sem)
    cp_i = pltpu.async_copy(if_hbm.at[iid_v], i_rows, sem)
    cp_ub = pltpu.async_copy(ub_hbm.at[uid_v], ub_v, sem)
    cp_ib = pltpu.async_copy(ib_hbm.at[iid_v], ib_v, sem)
    cp_u.wait()
    cp_i.wait()
    cp_ub.wait()
    cp_ib.wait()

    gb_vec = gb_v[...]

    def block(j, carry):
        b16 = j * 16
        row_idx = lax.iota(jnp.int32, 16) + b16
        acc = ub_v[pl.ds(b16, 16)] + ib_v[pl.ds(b16, 16)] + gb_vec
        for d in range(EMBED):
            col = jnp.full((16,), d, jnp.int32)
            uu = plsc.load_gather(u_rows, [row_idx, col])
            ii = plsc.load_gather(i_rows, [row_idx, col])
            acc = acc + uu * ii
        pred_v[pl.ds(b16, 16)] = acc
        cvr_v[pl.ds(b16, 16)] = 1.0 / (1.0 + jnp.exp(-acc))
        return carry

    lax.fori_loop(0, _BLOCKS, block, 0)

    pltpu.sync_copy(pred_v, pred_hbm.at[pl.ds(base, _CHUNK)])
    pltpu.sync_copy(cvr_v, cvr_hbm.at[pl.ds(base, _CHUNK)])


@functools.partial(jax.jit, static_argnames=())
def _run(user_id, item_id, user_factors, item_factors, user_bias, item_bias,
         gb16):
    f32 = jnp.float32
    krn = pl.kernel(
        _body,
        out_type=(jax.ShapeDtypeStruct((BATCH,), f32),
                  jax.ShapeDtypeStruct((BATCH,), f32)),
        mesh=plsc.VectorSubcoreMesh(core_axis_name="c", subcore_axis_name="s"),
        compiler_params=pltpu.CompilerParams(
            needs_layout_passes=False, use_tc_tiling_on_sc=False),
        scratch_types=[
            pltpu.VMEM((_CHUNK,), jnp.int32),
            pltpu.VMEM((_CHUNK,), jnp.int32),
            pltpu.VMEM((_CHUNK, EMBED), f32),
            pltpu.VMEM((_CHUNK, EMBED), f32),
            pltpu.VMEM((_CHUNK,), f32),
            pltpu.VMEM((_CHUNK,), f32),
            pltpu.VMEM((_CHUNK,), f32),
            pltpu.VMEM((_CHUNK,), f32),
            pltpu.VMEM((16,), f32),
            pltpu.SemaphoreType.DMA,
        ],
    )
    return krn(user_id, item_id, user_factors, item_factors, user_bias,
               item_bias, gb16)


def kernel(user_id, item_id, user_factors, item_factors, user_bias,
           item_bias, global_bias):
    gb16 = jnp.broadcast_to(global_bias.astype(jnp.float32), (16,))
    pred, cvr = _run(user_id.astype(jnp.int32), item_id.astype(jnp.int32),
                     user_factors, item_factors, user_bias, item_bias, gb16)
    return (pred, cvr)
